# Initial kernel scaffold; baseline (speedup 1.0000x reference)
import jax
import jax.numpy as jnp
from jax.experimental import pallas as pl


def kernel(mid_feat_q, mid_feat_k, labels_q, labels_k, W1, b1, gamma, beta, W2, b2, Wlin, blin, neg_queue, pos_queue, neg_ptr, pos_ptr):
    raise NotImplementedError("write your pallas kernel here")


# speedup vs baseline: 24.6338x; 24.6338x over previous
import jax
import jax.numpy as jnp
from jax.experimental import pallas as pl
from jax.experimental.pallas import tpu as pltpu

B = 1024
DF = 2048
DIM = 128
C = 8192
P = 8
N = 4
BN_EPS = 1e-5

NEG_W = N * C
POS_W = P * C
SIM_W = B + P + N * (C - 1)
BLK = 512
NCB = (SIM_W + BLK - 1) // BLK
PAD_OFF = B + P
PAD_W = NCB * BLK + 128

_f32 = jnp.float32


def _mlp_body(x_ref, w1_ref, b1_ref, g_ref, be_ref, w2_ref, b2_ref, out_ref):
    j = pl.program_id(1)
    x = x_ref[0]
    h = jnp.dot(x, w1_ref[...], preferred_element_type=_f32) + b1_ref[...]
    mu = jnp.mean(h, axis=0, keepdims=True)
    hc = h - mu
    var = jnp.mean(hc * hc, axis=0, keepdims=True)
    hb = hc * jax.lax.rsqrt(var + BN_EPS) * g_ref[...] + be_ref[...]
    hb = jnp.maximum(hb, 0.0)
    p = jnp.dot(hb, w2_ref[...], preferred_element_type=_f32)

    @pl.when(j == 0)
    def _():
        out_ref[0] = p + b2_ref[...]

    @pl.when(j > 0)
    def _():
        out_ref[0] = out_ref[0] + p

    @pl.when(j == pl.num_programs(1) - 1)
    def _():
        f = out_ref[0]
        nrm = jnp.sqrt(jnp.sum(f * f, axis=1, keepdims=True))
        out_ref[0] = f / jnp.maximum(nrm, 1e-12)


def _logits_body(xq_ref, xk_ref, w_ref, b_ref, oq_ref, ok_ref):
    w = w_ref[...]
    b = b_ref[...]
    oq_ref[...] = jnp.dot(xq_ref[...], w, preferred_element_type=_f32) + b
    ok_ref[...] = jnp.dot(xk_ref[...], w, preferred_element_type=_f32) + b


def _poslist_body(pq_ref, fq_ref, lq_ref, out_ref):
    i = pl.program_id(0)
    j = pl.program_id(1)
    blk = i * 64 + j
    zt = jax.lax.dot_general(pq_ref[...], fq_ref[...], (((0,), (1,)), ((), ())),
                             preferred_element_type=_f32)
    zt3 = zt.reshape(BLK // P, P, B)
    cls = blk * (BLK // P) + jax.lax.broadcasted_iota(jnp.int32, (BLK // P, 1, B), 0)
    m = cls == lq_ref[...].reshape(1, 1, B)
    contrib = jnp.sum(jnp.where(m, zt3, 0.0), axis=0)

    @pl.when(j == 0)
    def _():
        out_ref[0] = contrib

    @pl.when(j > 0)
    def _():
        out_ref[0] = out_ref[0] + contrib


def _prep_body(lk_ref, lkc_ref, nptr_ref, pptr_ref,
               dn_ref, an_ref, dp_ref, ap_ref, nptr2_ref, pptr2_ref):
    l_row = lk_ref[...]
    ii = jax.lax.broadcasted_iota(jnp.int32, (B, B), 0)
    jj = jax.lax.broadcasted_iota(jnp.int32, (B, B), 1)
    eq = lkc_ref[...] == l_row
    eqf = jnp.where(eq, 1.0, 0.0)
    rank = jnp.sum(jnp.where(ii < jj, eqf, 0.0), axis=0, keepdims=True)
    cnt = jnp.sum(eqf, axis=0, keepdims=True)
    rank = rank.astype(jnp.int32)
    cnt = cnt.astype(jnp.int32)

    nb = jnp.zeros((1, B), _f32)
    pb = jnp.zeros((1, B), _f32)
    for c0 in range(C // B):
        cls = c0 * B + jax.lax.broadcasted_iota(jnp.int32, (B, B), 0)
        oc = jnp.where(cls == l_row, 1.0, 0.0)
        npc = nptr_ref[pl.ds(c0 * B, B), :].astype(_f32)
        ppc = pptr_ref[pl.ds(c0 * B, B), :].astype(_f32)
        nb = nb + jnp.sum(oc * npc, axis=0, keepdims=True)
        pb = pb + jnp.sum(oc * ppc, axis=0, keepdims=True)
        cc = jnp.sum(oc, axis=1, keepdims=True).astype(jnp.int32)
        nptr2_ref[pl.ds(c0 * B, B), :] = (nptr_ref[pl.ds(c0 * B, B), :] + cc) & (N - 1)
        pptr2_ref[pl.ds(c0 * B, B), :] = (pptr_ref[pl.ds(c0 * B, B), :] + cc) & (P - 1)
    nb = nb.astype(jnp.int32)
    pb = pb.astype(jnp.int32)

    dn_ref[...] = l_row * N + ((nb + rank) & (N - 1))
    an_ref[...] = jnp.where(rank + N >= cnt, 1, 0)
    dp_ref[...] = l_row * P + ((pb + rank) & (P - 1))
    ap_ref[...] = jnp.where(rank + P >= cnt, 1, 0)


def _simcon_body(nqp_ref, fq_ref, fk_ref, lqc_ref, lkr_ref, pos_ref,
                 sim_ref, lab_ref):
    j = pl.program_id(0)
    fq = fq_ref[...]
    cols = j * BLK + jax.lax.broadcasted_iota(jnp.int32, (B, BLK), 1)

    @pl.when(j < 2)
    def _():
        ks = fk_ref[pl.ds(j * BLK, BLK), :]
        sim_ref[...] = jax.lax.dot_general(fq, ks, (((1,), (1,)), ((), ())),
                                           preferred_element_type=_f32)
        lk = lkr_ref[:, pl.ds(j * BLK, BLK)]
        lab_ref[...] = jnp.where(lqc_ref[...] == lk, 1.0, 0.0)

    @pl.when(j >= 2)
    def _():
        s = nqp_ref[:, pl.ds(j * BLK, BLK + 128)]
        full = jnp.dot(fq, s, preferred_element_type=_f32)
        a = full[:, 0:BLK]
        bshift = full[:, N:BLK + N]
        thr = PAD_OFF + N * lqc_ref[...]
        out = jnp.where(cols < thr, a, bshift)
        e = jnp.where(jax.lax.broadcasted_iota(jnp.int32, (P, BLK), 0)
                      == jax.lax.broadcasted_iota(jnp.int32, (P, BLK), 1), 1.0, 0.0)
        pos_pad = jnp.dot(pos_ref[...], e, preferred_element_type=_f32)
        is_pos = (cols >= B) & (cols < PAD_OFF)
        sim_ref[...] = jnp.where(is_pos, pos_pad, out)
        lab_ref[...] = jnp.where(is_pos, 1.0, 0.0)


def _scatter_body(q_ref, fk_ref, dest_ref, alive_ref, out_ref):
    j = pl.program_id(0)
    colg = j * BLK + jax.lax.broadcasted_iota(jnp.int32, (BLK, B), 0)
    oh = jnp.where((colg == dest_ref[...]) & (alive_ref[...] > 0), 1.0, 0.0)
    upd = jax.lax.dot_general(fk_ref[...], oh, (((0,), (1,)), ((), ())),
                              preferred_element_type=_f32)
    cov = jax.lax.dot_general(jnp.ones((1, B), _f32), oh, (((1,), (1,)), ((), ())),
                              preferred_element_type=_f32)
    out_ref[...] = jnp.where(cov > 0.5, upd, q_ref[...])


def kernel(mid_feat_q, mid_feat_k, labels_q, labels_k,
           W1, b1, gamma, beta, W2, b2, Wlin, blin,
           neg_queue, pos_queue, neg_ptr, pos_ptr):
    x = jnp.stack([mid_feat_q, mid_feat_k])
    b1r = b1.reshape(1, DF)
    gr = gamma.reshape(1, DF)
    ber = beta.reshape(1, DF)
    b2r = b2.reshape(1, DIM)
    blinr = blin.reshape(1, C)
    lq_row = labels_q.reshape(1, B)
    lq_col = labels_q.reshape(B, 1)
    lk_row = labels_k.reshape(1, B)
    nptr_col = neg_ptr.reshape(C, 1)
    pptr_col = pos_ptr.reshape(C, 1)

    dfb = DF // 8
    feat = pl.pallas_call(
        _mlp_body,
        grid=(2, 8),
        in_specs=[
            pl.BlockSpec((1, B, DF), lambda i, j: (i, 0, 0)),
            pl.BlockSpec((DF, dfb), lambda i, j: (0, j)),
            pl.BlockSpec((1, dfb), lambda i, j: (0, j)),
            pl.BlockSpec((1, dfb), lambda i, j: (0, j)),
            pl.BlockSpec((1, dfb), lambda i, j: (0, j)),
            pl.BlockSpec((dfb, DIM), lambda i, j: (j, 0)),
            pl.BlockSpec((1, DIM), lambda i, j: (0, 0)),
        ],
        out_specs=pl.BlockSpec((1, B, DIM), lambda i, j: (i, 0, 0)),
        out_shape=jax.ShapeDtypeStruct((2, B, DIM), _f32),
        compiler_params=pltpu.CompilerParams(
            dimension_semantics=("parallel", "arbitrary")),
    )(x, W1, b1r, gr, ber, W2, b2r)
    feat_q = feat[0]
    feat_k = feat[1]

    logit_q, logit_k = pl.pallas_call(
        _logits_body,
        grid=(16,),
        in_specs=[
            pl.BlockSpec((B, DF), lambda j: (0, 0)),
            pl.BlockSpec((B, DF), lambda j: (0, 0)),
            pl.BlockSpec((DF, BLK), lambda j: (0, j)),
            pl.BlockSpec((1, BLK), lambda j: (0, j)),
        ],
        out_specs=[pl.BlockSpec((B, BLK), lambda j: (0, j))] * 2,
        out_shape=[jax.ShapeDtypeStruct((B, C), _f32)] * 2,
        compiler_params=pltpu.CompilerParams(
            dimension_semantics=("parallel",)),
    )(mid_feat_q, mid_feat_k, Wlin, blinr)

    plt = pl.pallas_call(
        _poslist_body,
        grid=(2, 64),
        in_specs=[
            pl.BlockSpec((DIM, BLK), lambda i, j: (0, i * 64 + j)),
            pl.BlockSpec((B, DIM), lambda i, j: (0, 0)),
            pl.BlockSpec((1, B), lambda i, j: (0, 0)),
        ],
        out_specs=pl.BlockSpec((1, P, B), lambda i, j: (i, 0, 0)),
        out_shape=jax.ShapeDtypeStruct((2, P, B), _f32),
        compiler_params=pltpu.CompilerParams(
            dimension_semantics=("parallel", "arbitrary")),
    )(pos_queue, feat_q, lq_row)
    pos_list = jnp.transpose(plt[0] + plt[1])

    dn, an, dp, ap, nptr2, pptr2 = pl.pallas_call(
        _prep_body,
        grid=(1,),
        in_specs=[
            pl.BlockSpec((1, B), lambda i: (0, 0)),
            pl.BlockSpec((B, 1), lambda i: (0, 0)),
            pl.BlockSpec((C, 1), lambda i: (0, 0)),
            pl.BlockSpec((C, 1), lambda i: (0, 0)),
        ],
        out_specs=[
            pl.BlockSpec((1, B), lambda i: (0, 0)),
            pl.BlockSpec((1, B), lambda i: (0, 0)),
            pl.BlockSpec((1, B), lambda i: (0, 0)),
            pl.BlockSpec((1, B), lambda i: (0, 0)),
            pl.BlockSpec((C, 1), lambda i: (0, 0)),
            pl.BlockSpec((C, 1), lambda i: (0, 0)),
        ],
        out_shape=[
            jax.ShapeDtypeStruct((1, B), jnp.int32),
            jax.ShapeDtypeStruct((1, B), jnp.int32),
            jax.ShapeDtypeStruct((1, B), jnp.int32),
            jax.ShapeDtypeStruct((1, B), jnp.int32),
            jax.ShapeDtypeStruct((C, 1), jnp.int32),
            jax.ShapeDtypeStruct((C, 1), jnp.int32),
        ],
        compiler_params=pltpu.CompilerParams(
            dimension_semantics=("arbitrary",)),
    )(lk_row, labels_k.reshape(B, 1), nptr_col, pptr_col)

    nqp = jnp.pad(neg_queue, ((0, 0), (PAD_OFF, PAD_W - PAD_OFF - NEG_W)))
    sim_con, labels_con = pl.pallas_call(
        _simcon_body,
        grid=(NCB,),
        in_specs=[
            pl.BlockSpec((DIM, PAD_W), lambda j: (0, 0)),
            pl.BlockSpec((B, DIM), lambda j: (0, 0)),
            pl.BlockSpec((B, DIM), lambda j: (0, 0)),
            pl.BlockSpec((B, 1), lambda j: (0, 0)),
            pl.BlockSpec((1, B), lambda j: (0, 0)),
            pl.BlockSpec((B, P), lambda j: (0, 0)),
        ],
        out_specs=[pl.BlockSpec((B, BLK), lambda j: (0, j))] * 2,
        out_shape=[jax.ShapeDtypeStruct((B, SIM_W), _f32)] * 2,
        compiler_params=pltpu.CompilerParams(
            dimension_semantics=("parallel",)),
    )(nqp, feat_q, feat_k, lq_col, lk_row, pos_list)

    def scatter_call(queue, width, dest, alive):
        return pl.pallas_call(
            _scatter_body,
            grid=(width // BLK,),
            in_specs=[
                pl.BlockSpec((DIM, BLK), lambda j: (0, j)),
                pl.BlockSpec((B, DIM), lambda j: (0, 0)),
                pl.BlockSpec((1, B), lambda j: (0, 0)),
                pl.BlockSpec((1, B), lambda j: (0, 0)),
            ],
            out_specs=pl.BlockSpec((DIM, BLK), lambda j: (0, j)),
            out_shape=jax.ShapeDtypeStruct((DIM, width), _f32),
            compiler_params=pltpu.CompilerParams(
                dimension_semantics=("parallel",)),
        )(queue, feat_k, dest, alive)

    neg_queue2 = scatter_call(neg_queue, NEG_W, dn, an)
    pos_queue2 = scatter_call(pos_queue, POS_W, dp, ap)

    return (sim_con, labels_con, logit_q, logit_k,
            neg_queue2, pos_queue2,
            nptr2.reshape(C), pptr2.reshape(C))


# speedup vs baseline: 25.9917x; 1.0551x over previous
import jax
import jax.numpy as jnp
from jax.experimental import pallas as pl
from jax.experimental.pallas import tpu as pltpu

B = 1024
DF = 2048
DIM = 128
C = 8192
P = 8
N = 4
BN_EPS = 1e-5

NEG_W = N * C
POS_W = P * C
SIM_W = B + P + N * (C - 1)
BLK = 512
SBLK = 1024
QBLK = 1024
PBLK = 1024
NCB = (SIM_W + SBLK - 1) // SBLK
PAD_OFF = B + P
PAD_W = NCB * SBLK + 128

_f32 = jnp.float32


def _mlp_body(x_ref, w1_ref, b1_ref, g_ref, be_ref, w2_ref, b2_ref, out_ref):
    j = pl.program_id(1)
    x = x_ref[0]
    h = jnp.dot(x, w1_ref[...], preferred_element_type=_f32) + b1_ref[...]
    mu = jnp.mean(h, axis=0, keepdims=True)
    hc = h - mu
    var = jnp.mean(hc * hc, axis=0, keepdims=True)
    hb = hc * jax.lax.rsqrt(var + BN_EPS) * g_ref[...] + be_ref[...]
    hb = jnp.maximum(hb, 0.0)
    p = jnp.dot(hb, w2_ref[...], preferred_element_type=_f32)

    @pl.when(j == 0)
    def _():
        out_ref[0] = p + b2_ref[...]

    @pl.when(j > 0)
    def _():
        out_ref[0] = out_ref[0] + p

    @pl.when(j == pl.num_programs(1) - 1)
    def _():
        f = out_ref[0]
        nrm = jnp.sqrt(jnp.sum(f * f, axis=1, keepdims=True))
        out_ref[0] = f / jnp.maximum(nrm, 1e-12)


def _logits_body(xq_ref, xk_ref, w_ref, b_ref, oq_ref, ok_ref):
    w = w_ref[...]
    b = b_ref[...]
    oq_ref[...] = jnp.dot(xq_ref[...], w, preferred_element_type=_f32) + b
    ok_ref[...] = jnp.dot(xk_ref[...], w, preferred_element_type=_f32) + b


def _poslist_body(pq_ref, fq_ref, lq_ref, out_ref):
    i = pl.program_id(0)
    j = pl.program_id(1)
    nj = pl.num_programs(1)
    blk = i * nj + j
    zt = jax.lax.dot_general(pq_ref[...], fq_ref[...], (((0,), (1,)), ((), ())),
                             preferred_element_type=_f32)
    zt3 = zt.reshape(PBLK // P, P, B)
    cls = blk * (PBLK // P) + jax.lax.broadcasted_iota(jnp.int32, (PBLK // P, 1, B), 0)
    m = cls == lq_ref[...].reshape(1, 1, B)
    contrib = jnp.sum(jnp.where(m, zt3, 0.0), axis=0)

    @pl.when(j == 0)
    def _():
        out_ref[0] = contrib

    @pl.when(j > 0)
    def _():
        out_ref[0] = out_ref[0] + contrib


def _prep_body(lk_ref, lkc_ref, nptr_ref, pptr_ref,
               dn_ref, an_ref, dp_ref, ap_ref, nptr2_ref, pptr2_ref):
    l_row = lk_ref[...]
    ii = jax.lax.broadcasted_iota(jnp.int32, (B, B), 0)
    jj = jax.lax.broadcasted_iota(jnp.int32, (B, B), 1)
    eq = lkc_ref[...] == l_row
    eqf = jnp.where(eq, 1.0, 0.0)
    rank = jnp.sum(jnp.where(ii < jj, eqf, 0.0), axis=0, keepdims=True)
    cnt = jnp.sum(eqf, axis=0, keepdims=True)
    rank = rank.astype(jnp.int32)
    cnt = cnt.astype(jnp.int32)

    nb = jnp.zeros((1, B), _f32)
    pb = jnp.zeros((1, B), _f32)
    for c0 in range(C // B):
        cls = c0 * B + jax.lax.broadcasted_iota(jnp.int32, (B, B), 0)
        oc = jnp.where(cls == l_row, 1.0, 0.0)
        npc = nptr_ref[pl.ds(c0 * B, B), :].astype(_f32)
        ppc = pptr_ref[pl.ds(c0 * B, B), :].astype(_f32)
        nb = nb + jnp.sum(oc * npc, axis=0, keepdims=True)
        pb = pb + jnp.sum(oc * ppc, axis=0, keepdims=True)
        cc = jnp.sum(oc, axis=1, keepdims=True).astype(jnp.int32)
        nptr2_ref[pl.ds(c0 * B, B), :] = (nptr_ref[pl.ds(c0 * B, B), :] + cc) & (N - 1)
        pptr2_ref[pl.ds(c0 * B, B), :] = (pptr_ref[pl.ds(c0 * B, B), :] + cc) & (P - 1)
    nb = nb.astype(jnp.int32)
    pb = pb.astype(jnp.int32)

    dn_ref[...] = l_row * N + ((nb + rank) & (N - 1))
    an_ref[...] = jnp.where(rank + N >= cnt, 1, 0)
    dp_ref[...] = l_row * P + ((pb + rank) & (P - 1))
    ap_ref[...] = jnp.where(rank + P >= cnt, 1, 0)


def _simcon_body(nqp_ref, fq_ref, fk_ref, lqc_ref, lkr_ref, pos_ref,
                 sim_ref, lab_ref):
    j = pl.program_id(0)
    fq = fq_ref[...]
    cols = j * SBLK + jax.lax.broadcasted_iota(jnp.int32, (B, SBLK), 1)

    @pl.when(j == 0)
    def _():
        sim_ref[...] = jax.lax.dot_general(fq, fk_ref[...], (((1,), (1,)), ((), ())),
                                           preferred_element_type=_f32)
        lab_ref[...] = jnp.where(lqc_ref[...] == lkr_ref[...], 1.0, 0.0)

    @pl.when(j >= 1)
    def _():
        s = nqp_ref[:, pl.ds(j * SBLK, SBLK + 128)]
        full = jnp.dot(fq, s, preferred_element_type=_f32)
        a = full[:, 0:SBLK]
        bshift = full[:, N:SBLK + N]
        thr = PAD_OFF + N * lqc_ref[...]
        out = jnp.where(cols < thr, a, bshift)
        e = jnp.where(jax.lax.broadcasted_iota(jnp.int32, (P, SBLK), 0)
                      == jax.lax.broadcasted_iota(jnp.int32, (P, SBLK), 1), 1.0, 0.0)
        pos_pad = jnp.dot(pos_ref[...], e, preferred_element_type=_f32)
        is_pos = (cols >= B) & (cols < PAD_OFF)
        sim_ref[...] = jnp.where(is_pos, pos_pad, out)
        lab_ref[...] = jnp.where(is_pos, 1.0, 0.0)


def _scatter_body(q_ref, fk_ref, dest_ref, alive_ref, out_ref):
    j = pl.program_id(0)
    colg = j * QBLK + jax.lax.broadcasted_iota(jnp.int32, (QBLK, B), 0)
    oh = jnp.where((colg == dest_ref[...]) & (alive_ref[...] > 0), 1.0, 0.0)
    upd = jax.lax.dot_general(fk_ref[...], oh, (((0,), (1,)), ((), ())),
                              preferred_element_type=_f32)
    cov = jax.lax.dot_general(jnp.ones((1, B), _f32), oh, (((1,), (1,)), ((), ())),
                              preferred_element_type=_f32)
    out_ref[...] = jnp.where(cov > 0.5, upd, q_ref[...])


def kernel(mid_feat_q, mid_feat_k, labels_q, labels_k,
           W1, b1, gamma, beta, W2, b2, Wlin, blin,
           neg_queue, pos_queue, neg_ptr, pos_ptr):
    x = jnp.stack([mid_feat_q, mid_feat_k])
    b1r = b1.reshape(1, DF)
    gr = gamma.reshape(1, DF)
    ber = beta.reshape(1, DF)
    b2r = b2.reshape(1, DIM)
    blinr = blin.reshape(1, C)
    lq_row = labels_q.reshape(1, B)
    lq_col = labels_q.reshape(B, 1)
    lk_row = labels_k.reshape(1, B)
    nptr_col = neg_ptr.reshape(C, 1)
    pptr_col = pos_ptr.reshape(C, 1)

    dfb = DF // 8
    feat = pl.pallas_call(
        _mlp_body,
        grid=(2, 8),
        in_specs=[
            pl.BlockSpec((1, B, DF), lambda i, j: (i, 0, 0)),
            pl.BlockSpec((DF, dfb), lambda i, j: (0, j)),
            pl.BlockSpec((1, dfb), lambda i, j: (0, j)),
            pl.BlockSpec((1, dfb), lambda i, j: (0, j)),
            pl.BlockSpec((1, dfb), lambda i, j: (0, j)),
            pl.BlockSpec((dfb, DIM), lambda i, j: (j, 0)),
            pl.BlockSpec((1, DIM), lambda i, j: (0, 0)),
        ],
        out_specs=pl.BlockSpec((1, B, DIM), lambda i, j: (i, 0, 0)),
        out_shape=jax.ShapeDtypeStruct((2, B, DIM), _f32),
        compiler_params=pltpu.CompilerParams(
            dimension_semantics=("parallel", "arbitrary")),
    )(x, W1, b1r, gr, ber, W2, b2r)
    feat_q = feat[0]
    feat_k = feat[1]

    logit_q, logit_k = pl.pallas_call(
        _logits_body,
        grid=(16,),
        in_specs=[
            pl.BlockSpec((B, DF), lambda j: (0, 0)),
            pl.BlockSpec((B, DF), lambda j: (0, 0)),
            pl.BlockSpec((DF, BLK), lambda j: (0, j)),
            pl.BlockSpec((1, BLK), lambda j: (0, j)),
        ],
        out_specs=[pl.BlockSpec((B, BLK), lambda j: (0, j))] * 2,
        out_shape=[jax.ShapeDtypeStruct((B, C), _f32)] * 2,
        compiler_params=pltpu.CompilerParams(
            dimension_semantics=("parallel",)),
    )(mid_feat_q, mid_feat_k, Wlin, blinr)

    npb = POS_W // PBLK // 2
    plt = pl.pallas_call(
        _poslist_body,
        grid=(2, npb),
        in_specs=[
            pl.BlockSpec((DIM, PBLK), lambda i, j: (0, i * npb + j)),
            pl.BlockSpec((B, DIM), lambda i, j: (0, 0)),
            pl.BlockSpec((1, B), lambda i, j: (0, 0)),
        ],
        out_specs=pl.BlockSpec((1, P, B), lambda i, j: (i, 0, 0)),
        out_shape=jax.ShapeDtypeStruct((2, P, B), _f32),
        compiler_params=pltpu.CompilerParams(
            dimension_semantics=("parallel", "arbitrary")),
    )(pos_queue, feat_q, lq_row)
    pos_list = jnp.transpose(plt[0] + plt[1])

    dn, an, dp, ap, nptr2, pptr2 = pl.pallas_call(
        _prep_body,
        grid=(1,),
        in_specs=[
            pl.BlockSpec((1, B), lambda i: (0, 0)),
            pl.BlockSpec((B, 1), lambda i: (0, 0)),
            pl.BlockSpec((C, 1), lambda i: (0, 0)),
            pl.BlockSpec((C, 1), lambda i: (0, 0)),
        ],
        out_specs=[
            pl.BlockSpec((1, B), lambda i: (0, 0)),
            pl.BlockSpec((1, B), lambda i: (0, 0)),
            pl.BlockSpec((1, B), lambda i: (0, 0)),
            pl.BlockSpec((1, B), lambda i: (0, 0)),
            pl.BlockSpec((C, 1), lambda i: (0, 0)),
            pl.BlockSpec((C, 1), lambda i: (0, 0)),
        ],
        out_shape=[
            jax.ShapeDtypeStruct((1, B), jnp.int32),
            jax.ShapeDtypeStruct((1, B), jnp.int32),
            jax.ShapeDtypeStruct((1, B), jnp.int32),
            jax.ShapeDtypeStruct((1, B), jnp.int32),
            jax.ShapeDtypeStruct((C, 1), jnp.int32),
            jax.ShapeDtypeStruct((C, 1), jnp.int32),
        ],
        compiler_params=pltpu.CompilerParams(
            dimension_semantics=("arbitrary",)),
    )(lk_row, labels_k.reshape(B, 1), nptr_col, pptr_col)

    nqp = jnp.pad(neg_queue, ((0, 0), (PAD_OFF, PAD_W - PAD_OFF - NEG_W)))
    sim_con, labels_con = pl.pallas_call(
        _simcon_body,
        grid=(NCB,),
        in_specs=[
            pl.BlockSpec((DIM, PAD_W), lambda j: (0, 0)),
            pl.BlockSpec((B, DIM), lambda j: (0, 0)),
            pl.BlockSpec((B, DIM), lambda j: (0, 0)),
            pl.BlockSpec((B, 1), lambda j: (0, 0)),
            pl.BlockSpec((1, B), lambda j: (0, 0)),
            pl.BlockSpec((B, P), lambda j: (0, 0)),
        ],
        out_specs=[pl.BlockSpec((B, SBLK), lambda j: (0, j))] * 2,
        out_shape=[jax.ShapeDtypeStruct((B, SIM_W), _f32)] * 2,
        compiler_params=pltpu.CompilerParams(
            dimension_semantics=("parallel",)),
    )(nqp, feat_q, feat_k, lq_col, lk_row, pos_list)

    def scatter_call(queue, width, dest, alive):
        return pl.pallas_call(
            _scatter_body,
            grid=(width // QBLK,),
            in_specs=[
                pl.BlockSpec((DIM, QBLK), lambda j: (0, j)),
                pl.BlockSpec((B, DIM), lambda j: (0, 0)),
                pl.BlockSpec((1, B), lambda j: (0, 0)),
                pl.BlockSpec((1, B), lambda j: (0, 0)),
            ],
            out_specs=pl.BlockSpec((DIM, QBLK), lambda j: (0, j)),
            out_shape=jax.ShapeDtypeStruct((DIM, width), _f32),
            compiler_params=pltpu.CompilerParams(
                dimension_semantics=("parallel",)),
        )(queue, feat_k, dest, alive)

    neg_queue2 = scatter_call(neg_queue, NEG_W, dn, an)
    pos_queue2 = scatter_call(pos_queue, POS_W, dp, ap)

    return (sim_con, labels_con, logit_q, logit_k,
            neg_queue2, pos_queue2,
            nptr2.reshape(C), pptr2.reshape(C))


# speedup vs baseline: 26.2588x; 1.0103x over previous
import jax
import jax.numpy as jnp
from jax.experimental import pallas as pl
from jax.experimental.pallas import tpu as pltpu

B = 1024
DF = 2048
DIM = 128
C = 8192
P = 8
N = 4
BN_EPS = 1e-5

NEG_W = N * C
POS_W = P * C
SIM_W = B + P + N * (C - 1)
BLK = 512
SBLK = 1024
QBLK = 1024
PBLK = 1024
NCB = (SIM_W + SBLK - 1) // SBLK
PAD_OFF = B + P
PAD_W = NCB * SBLK + 128

_f32 = jnp.float32


def _mlp_body(x_ref, w1_ref, b1_ref, g_ref, be_ref, w2_ref, b2_ref, out_ref):
    j = pl.program_id(1)
    x = x_ref[0]
    h = jnp.dot(x, w1_ref[...], preferred_element_type=_f32) + b1_ref[...]
    mu = jnp.mean(h, axis=0, keepdims=True)
    hc = h - mu
    var = jnp.mean(hc * hc, axis=0, keepdims=True)
    hb = hc * jax.lax.rsqrt(var + BN_EPS) * g_ref[...] + be_ref[...]
    hb = jnp.maximum(hb, 0.0)
    p = jnp.dot(hb, w2_ref[...], preferred_element_type=_f32)

    @pl.when(j == 0)
    def _():
        out_ref[0] = p + b2_ref[...]

    @pl.when(j > 0)
    def _():
        out_ref[0] = out_ref[0] + p

    @pl.when(j == pl.num_programs(1) - 1)
    def _():
        f = out_ref[0]
        nrm = jnp.sqrt(jnp.sum(f * f, axis=1, keepdims=True))
        out_ref[0] = f / jnp.maximum(nrm, 1e-12)


def _logits_body(xq_ref, xk_ref, w_ref, b_ref, oq_ref, ok_ref):
    w = w_ref[...].astype(jnp.bfloat16)
    b = b_ref[...]
    xq = xq_ref[...].astype(jnp.bfloat16)
    xk = xk_ref[...].astype(jnp.bfloat16)
    oq_ref[...] = jnp.dot(xq, w, preferred_element_type=_f32) + b
    ok_ref[...] = jnp.dot(xk, w, preferred_element_type=_f32) + b


def _poslist_body(pq_ref, fq_ref, lq_ref, out_ref):
    i = pl.program_id(0)
    j = pl.program_id(1)
    nj = pl.num_programs(1)
    blk = i * nj + j
    zt = jax.lax.dot_general(pq_ref[...], fq_ref[...], (((0,), (1,)), ((), ())),
                             preferred_element_type=_f32)
    zt3 = zt.reshape(PBLK // P, P, B)
    cls = blk * (PBLK // P) + jax.lax.broadcasted_iota(jnp.int32, (PBLK // P, 1, B), 0)
    m = cls == lq_ref[...].reshape(1, 1, B)
    contrib = jnp.sum(jnp.where(m, zt3, 0.0), axis=0)

    @pl.when(j == 0)
    def _():
        out_ref[0] = contrib

    @pl.when(j > 0)
    def _():
        out_ref[0] = out_ref[0] + contrib


def _prep_body(lk_ref, lkc_ref, nptr_ref, pptr_ref,
               dn_ref, an_ref, dp_ref, ap_ref, nptr2_ref, pptr2_ref):
    l_row = lk_ref[...]
    ii = jax.lax.broadcasted_iota(jnp.int32, (B, B), 0)
    jj = jax.lax.broadcasted_iota(jnp.int32, (B, B), 1)
    eq = lkc_ref[...] == l_row
    eqf = jnp.where(eq, 1.0, 0.0)
    rank = jnp.sum(jnp.where(ii < jj, eqf, 0.0), axis=0, keepdims=True)
    cnt = jnp.sum(eqf, axis=0, keepdims=True)
    rank = rank.astype(jnp.int32)
    cnt = cnt.astype(jnp.int32)

    nb = jnp.zeros((1, B), _f32)
    pb = jnp.zeros((1, B), _f32)
    for c0 in range(C // B):
        cls = c0 * B + jax.lax.broadcasted_iota(jnp.int32, (B, B), 0)
        oc = jnp.where(cls == l_row, 1.0, 0.0)
        npc = nptr_ref[pl.ds(c0 * B, B), :].astype(_f32)
        ppc = pptr_ref[pl.ds(c0 * B, B), :].astype(_f32)
        nb = nb + jnp.sum(oc * npc, axis=0, keepdims=True)
        pb = pb + jnp.sum(oc * ppc, axis=0, keepdims=True)
        cc = jnp.sum(oc, axis=1, keepdims=True).astype(jnp.int32)
        nptr2_ref[pl.ds(c0 * B, B), :] = (nptr_ref[pl.ds(c0 * B, B), :] + cc) & (N - 1)
        pptr2_ref[pl.ds(c0 * B, B), :] = (pptr_ref[pl.ds(c0 * B, B), :] + cc) & (P - 1)
    nb = nb.astype(jnp.int32)
    pb = pb.astype(jnp.int32)

    dn_ref[...] = l_row * N + ((nb + rank) & (N - 1))
    an_ref[...] = jnp.where(rank + N >= cnt, 1, 0)
    dp_ref[...] = l_row * P + ((pb + rank) & (P - 1))
    ap_ref[...] = jnp.where(rank + P >= cnt, 1, 0)


def _simcon_body(nqp_ref, fq_ref, fk_ref, lqc_ref, lkr_ref, pos_ref,
                 sim_ref, lab_ref):
    j = pl.program_id(0)
    fq = fq_ref[...]
    cols = j * SBLK + jax.lax.broadcasted_iota(jnp.int32, (B, SBLK), 1)

    @pl.when(j == 0)
    def _():
        sim_ref[...] = jax.lax.dot_general(fq, fk_ref[...], (((1,), (1,)), ((), ())),
                                           preferred_element_type=_f32)
        lab_ref[...] = jnp.where(lqc_ref[...] == lkr_ref[...], 1.0, 0.0)

    @pl.when(j >= 1)
    def _():
        s = nqp_ref[:, pl.ds(j * SBLK, SBLK + 128)]
        full = jnp.dot(fq, s, preferred_element_type=_f32)
        a = full[:, 0:SBLK]
        bshift = full[:, N:SBLK + N]
        thr = PAD_OFF + N * lqc_ref[...]
        out = jnp.where(cols < thr, a, bshift)
        e = jnp.where(jax.lax.broadcasted_iota(jnp.int32, (P, SBLK), 0)
                      == jax.lax.broadcasted_iota(jnp.int32, (P, SBLK), 1), 1.0, 0.0)
        pos_pad = jnp.dot(pos_ref[...], e, preferred_element_type=_f32)
        is_pos = (cols >= B) & (cols < PAD_OFF)
        sim_ref[...] = jnp.where(is_pos, pos_pad, out)
        lab_ref[...] = jnp.where(is_pos, 1.0, 0.0)


def _scatter_body(q_ref, fk_ref, dest_ref, alive_ref, out_ref):
    j = pl.program_id(0)
    colg = j * QBLK + jax.lax.broadcasted_iota(jnp.int32, (QBLK, B), 0)
    oh = jnp.where((colg == dest_ref[...]) & (alive_ref[...] > 0), 1.0, 0.0)
    upd = jax.lax.dot_general(fk_ref[...], oh, (((0,), (1,)), ((), ())),
                              preferred_element_type=_f32)
    cov = jax.lax.dot_general(jnp.ones((1, B), _f32), oh, (((1,), (1,)), ((), ())),
                              preferred_element_type=_f32)
    out_ref[...] = jnp.where(cov > 0.5, upd, q_ref[...])


def kernel(mid_feat_q, mid_feat_k, labels_q, labels_k,
           W1, b1, gamma, beta, W2, b2, Wlin, blin,
           neg_queue, pos_queue, neg_ptr, pos_ptr):
    x = jnp.stack([mid_feat_q, mid_feat_k])
    b1r = b1.reshape(1, DF)
    gr = gamma.reshape(1, DF)
    ber = beta.reshape(1, DF)
    b2r = b2.reshape(1, DIM)
    blinr = blin.reshape(1, C)
    lq_row = labels_q.reshape(1, B)
    lq_col = labels_q.reshape(B, 1)
    lk_row = labels_k.reshape(1, B)
    nptr_col = neg_ptr.reshape(C, 1)
    pptr_col = pos_ptr.reshape(C, 1)

    dfb = DF // 8
    feat = pl.pallas_call(
        _mlp_body,
        grid=(2, 8),
        in_specs=[
            pl.BlockSpec((1, B, DF), lambda i, j: (i, 0, 0)),
            pl.BlockSpec((DF, dfb), lambda i, j: (0, j)),
            pl.BlockSpec((1, dfb), lambda i, j: (0, j)),
            pl.BlockSpec((1, dfb), lambda i, j: (0, j)),
            pl.BlockSpec((1, dfb), lambda i, j: (0, j)),
            pl.BlockSpec((dfb, DIM), lambda i, j: (j, 0)),
            pl.BlockSpec((1, DIM), lambda i, j: (0, 0)),
        ],
        out_specs=pl.BlockSpec((1, B, DIM), lambda i, j: (i, 0, 0)),
        out_shape=jax.ShapeDtypeStruct((2, B, DIM), _f32),
        compiler_params=pltpu.CompilerParams(
            dimension_semantics=("parallel", "arbitrary")),
    )(x, W1, b1r, gr, ber, W2, b2r)
    feat_q = feat[0]
    feat_k = feat[1]

    logit_q, logit_k = pl.pallas_call(
        _logits_body,
        grid=(16,),
        in_specs=[
            pl.BlockSpec((B, DF), lambda j: (0, 0)),
            pl.BlockSpec((B, DF), lambda j: (0, 0)),
            pl.BlockSpec((DF, BLK), lambda j: (0, j)),
            pl.BlockSpec((1, BLK), lambda j: (0, j)),
        ],
        out_specs=[pl.BlockSpec((B, BLK), lambda j: (0, j))] * 2,
        out_shape=[jax.ShapeDtypeStruct((B, C), _f32)] * 2,
        compiler_params=pltpu.CompilerParams(
            dimension_semantics=("parallel",)),
    )(mid_feat_q, mid_feat_k, Wlin, blinr)

    npb = POS_W // PBLK // 2
    plt = pl.pallas_call(
        _poslist_body,
        grid=(2, npb),
        in_specs=[
            pl.BlockSpec((DIM, PBLK), lambda i, j: (0, i * npb + j)),
            pl.BlockSpec((B, DIM), lambda i, j: (0, 0)),
            pl.BlockSpec((1, B), lambda i, j: (0, 0)),
        ],
        out_specs=pl.BlockSpec((1, P, B), lambda i, j: (i, 0, 0)),
        out_shape=jax.ShapeDtypeStruct((2, P, B), _f32),
        compiler_params=pltpu.CompilerParams(
            dimension_semantics=("parallel", "arbitrary")),
    )(pos_queue, feat_q, lq_row)
    pos_list = jnp.transpose(plt[0] + plt[1])

    dn, an, dp, ap, nptr2, pptr2 = pl.pallas_call(
        _prep_body,
        grid=(1,),
        in_specs=[
            pl.BlockSpec((1, B), lambda i: (0, 0)),
            pl.BlockSpec((B, 1), lambda i: (0, 0)),
            pl.BlockSpec((C, 1), lambda i: (0, 0)),
            pl.BlockSpec((C, 1), lambda i: (0, 0)),
        ],
        out_specs=[
            pl.BlockSpec((1, B), lambda i: (0, 0)),
            pl.BlockSpec((1, B), lambda i: (0, 0)),
            pl.BlockSpec((1, B), lambda i: (0, 0)),
            pl.BlockSpec((1, B), lambda i: (0, 0)),
            pl.BlockSpec((C, 1), lambda i: (0, 0)),
            pl.BlockSpec((C, 1), lambda i: (0, 0)),
        ],
        out_shape=[
            jax.ShapeDtypeStruct((1, B), jnp.int32),
            jax.ShapeDtypeStruct((1, B), jnp.int32),
            jax.ShapeDtypeStruct((1, B), jnp.int32),
            jax.ShapeDtypeStruct((1, B), jnp.int32),
            jax.ShapeDtypeStruct((C, 1), jnp.int32),
            jax.ShapeDtypeStruct((C, 1), jnp.int32),
        ],
        compiler_params=pltpu.CompilerParams(
            dimension_semantics=("arbitrary",)),
    )(lk_row, labels_k.reshape(B, 1), nptr_col, pptr_col)

    nqp = jnp.pad(neg_queue, ((0, 0), (PAD_OFF, PAD_W - PAD_OFF - NEG_W)))
    sim_con, labels_con = pl.pallas_call(
        _simcon_body,
        grid=(NCB,),
        in_specs=[
            pl.BlockSpec((DIM, PAD_W), lambda j: (0, 0)),
            pl.BlockSpec((B, DIM), lambda j: (0, 0)),
            pl.BlockSpec((B, DIM), lambda j: (0, 0)),
            pl.BlockSpec((B, 1), lambda j: (0, 0)),
            pl.BlockSpec((1, B), lambda j: (0, 0)),
            pl.BlockSpec((B, P), lambda j: (0, 0)),
        ],
        out_specs=[pl.BlockSpec((B, SBLK), lambda j: (0, j))] * 2,
        out_shape=[jax.ShapeDtypeStruct((B, SIM_W), _f32)] * 2,
        compiler_params=pltpu.CompilerParams(
            dimension_semantics=("parallel",)),
    )(nqp, feat_q, feat_k, lq_col, lk_row, pos_list)

    def scatter_call(queue, width, dest, alive):
        return pl.pallas_call(
            _scatter_body,
            grid=(width // QBLK,),
            in_specs=[
                pl.BlockSpec((DIM, QBLK), lambda j: (0, j)),
                pl.BlockSpec((B, DIM), lambda j: (0, 0)),
                pl.BlockSpec((1, B), lambda j: (0, 0)),
                pl.BlockSpec((1, B), lambda j: (0, 0)),
            ],
            out_specs=pl.BlockSpec((DIM, QBLK), lambda j: (0, j)),
            out_shape=jax.ShapeDtypeStruct((DIM, width), _f32),
            compiler_params=pltpu.CompilerParams(
                dimension_semantics=("parallel",)),
        )(queue, feat_k, dest, alive)

    neg_queue2 = scatter_call(neg_queue, NEG_W, dn, an)
    pos_queue2 = scatter_call(pos_queue, POS_W, dp, ap)

    return (sim_con, labels_con, logit_q, logit_k,
            neg_queue2, pos_queue2,
            nptr2.reshape(C), pptr2.reshape(C))
